# baseline (speedup 1.0000x reference)
import jax
import jax.numpy as jnp
from jax import lax
from jax.experimental import pallas as pl
from jax.experimental.pallas import tpu as pltpu
from jax.experimental.pallas import tpu_sc as plsc

N = 50000
E = 800000
EP = 802816
ROWS = EP // 128
HID = 100
HIDP = 128

NW = 32
W_EDGES = EP // NW
GCH = 3136
G_NCHUNK = W_EDGES // GCH

TROWS = ROWS // 16
SK = 2
SCH = SK * 128
S_NCHUNK = TROWS // SK
NPAD = 50008
NT = N // 16
Q = 32

_sc_mesh = plsc.VectorSubcoreMesh(core_axis_name="c", subcore_axis_name="s")
_sc_params = pltpu.CompilerParams(use_tc_tiling_on_sc=False)


def _gather_body(x8, sg, rg, spos, rpos,
                 sidx0, ridx0, srows0, rrows0,
                 sidx1, ridx1, srows1, rrows1,
                 isem, gsem, wsem):
    c = lax.axis_index("c")
    s = lax.axis_index("s")
    wid = s * 2 + c
    edge_base = wid * W_EDGES
    bufs = ((sidx0, ridx0, srows0, rrows0), (sidx1, ridx1, srows1, rrows1))

    def load_idx(i, b):
        sidx, ridx, _, _ = bufs[b]
        e0 = edge_base + i * GCH
        pltpu.async_copy(sg.at[pl.ds(e0, GCH)], sidx, isem)
        pltpu.async_copy(rg.at[pl.ds(e0, GCH)], ridx, isem)

    def drain_idx(b):
        sidx, ridx, _, _ = bufs[b]
        pltpu.make_async_copy(sg.at[pl.ds(0, GCH)], sidx, isem).wait()
        pltpu.make_async_copy(rg.at[pl.ds(0, GCH)], ridx, isem).wait()

    def fire_gather(b):
        sidx, ridx, srows, rrows = bufs[b]
        pltpu.async_copy(x8.at[sidx], srows, gsem)
        pltpu.async_copy(x8.at[ridx], rrows, gsem)

    def drain_gather(b):
        sidx, ridx, srows, rrows = bufs[b]
        pltpu.make_async_copy(x8.at[sidx], srows, gsem).wait()
        pltpu.make_async_copy(x8.at[ridx], rrows, gsem).wait()

    def fire_wb(i, b):
        _, _, srows, rrows = bufs[b]
        e0 = edge_base + i * GCH
        pltpu.async_copy(srows, spos.at[pl.ds(e0, GCH)], wsem)
        pltpu.async_copy(rrows, rpos.at[pl.ds(e0, GCH)], wsem)

    def drain_wb(b):
        _, _, srows, rrows = bufs[b]
        pltpu.make_async_copy(srows, spos.at[pl.ds(0, GCH)], wsem).wait()
        pltpu.make_async_copy(rrows, rpos.at[pl.ds(0, GCH)], wsem).wait()

    load_idx(0, 0)

    def pair(p, carry):
        i = p * 2
        drain_idx(0)
        load_idx(i + 1, 1)

        @pl.when(p > 0)
        def _():
            drain_wb(0)
        fire_gather(0)
        drain_idx(1)
        drain_gather(0)
        fire_wb(i, 0)

        @pl.when(p > 0)
        def _():
            drain_wb(1)
        fire_gather(1)

        @pl.when(p + 1 < G_NCHUNK // 2)
        def _():
            load_idx(i + 2, 0)
        drain_gather(1)
        fire_wb(i + 1, 1)
        return carry

    lax.fori_loop(0, G_NCHUNK // 2, pair, 0)
    drain_wb(0)
    drain_wb(1)


def _scatter_body(e, ss, zeros, agg, acc,
                  sidx0, ebuf0, sidx1, ebuf1, lsem, csem):
    c = lax.axis_index("c")
    s = lax.axis_index("s")
    bufs = ((sidx0, ebuf0), (sidx1, ebuf1))

    def do_quarter(qoff):
        pltpu.sync_copy(zeros, acc.at[pl.ds(s * NT, NT)])
        plsc.subcore_barrier()

        def load(i, b):
            sidx, ebuf = bufs[b]
            r0 = s * TROWS + i * SK
            pltpu.async_copy(ss.at[pl.ds(r0, SK)], sidx, lsem)
            pltpu.async_copy(
                e.at[pl.ds(r0 * 128, SCH), pl.ds(qoff, Q)], ebuf, lsem)

        def drain_load(b):
            sidx, ebuf = bufs[b]
            pltpu.make_async_copy(ss.at[pl.ds(0, SK)], sidx, lsem).wait()
            pltpu.make_async_copy(
                e.at[pl.ds(0, SCH), pl.ds(qoff, Q)], ebuf, lsem).wait()

        def fire_scatter(b):
            sidx, ebuf = bufs[b]
            for j in range(SK):
                pltpu.async_copy(ebuf.at[pl.ds(j * 128, 128)],
                                 acc.at[sidx.at[j]], csem, add=True)

        def drain_scatter(b):
            sidx, ebuf = bufs[b]
            for j in range(SK):
                pltpu.make_async_copy(ebuf.at[pl.ds(j * 128, 128)],
                                      acc.at[sidx.at[0]], csem).wait()

        load(0, 0)

        def pair(p, carry):
            i = p * 2

            @pl.when(p > 0)
            def _():
                drain_scatter(1)
            load(i + 1, 1)
            drain_load(0)
            fire_scatter(0)
            drain_load(1)
            drain_scatter(0)

            @pl.when(p + 1 < S_NCHUNK // 2)
            def _():
                load(i + 2, 0)
            fire_scatter(1)
            return carry

        lax.fori_loop(0, S_NCHUNK // 2, pair, 0)
        drain_scatter(1)
        plsc.subcore_barrier()
        pltpu.sync_copy(acc.at[pl.ds(s * NT, NT)],
                        agg.at[pl.ds(s * NT, NT), pl.ds(qoff, Q)])
        plsc.subcore_barrier()

    @pl.when(c == 0)
    def _():
        do_quarter(0)
        do_quarter(2 * Q)

    @pl.when(c == 1)
    def _():
        do_quarter(Q)
        do_quarter(3 * Q)


def _relation_mlp_body(sp, rp, w1s, w1r, b1, w2, b2, w3, b3, w4, b4, out):
    f32 = jnp.float32
    bf16 = jnp.bfloat16
    h = jnp.dot(sp[...], w1s[...], preferred_element_type=f32)
    h = h + jnp.dot(rp[...], w1r[...], preferred_element_type=f32) + b1[...]
    h = jnp.maximum(h, 0.0)
    h = jnp.maximum(jnp.dot(h.astype(bf16), w2[...],
                            preferred_element_type=f32) + b2[...], 0.0)
    h = jnp.maximum(jnp.dot(h.astype(bf16), w3[...],
                            preferred_element_type=f32) + b3[...], 0.0)
    h = jnp.maximum(jnp.dot(h.astype(bf16), w4[...],
                            preferred_element_type=f32) + b4[...], 0.0)
    out[...] = h


def _object_mlp_body(x, agg, w1x, w1a, b1, w2, b2, out):
    f32 = jnp.float32
    h = jnp.dot(x[...], w1x[...], preferred_element_type=f32)
    h = h + jnp.dot(agg[...], w1a[...], preferred_element_type=f32) + b1[...]
    h = jnp.maximum(h, 0.0)
    out[...] = jnp.dot(h, w2[...], preferred_element_type=f32) + b2[...]


ET = 4096
E_GRID = EP // ET
NTC = 2000
N_GRID = N // NTC


def _full(shape):
    return pl.BlockSpec(shape, lambda i: (0,) * len(shape))


def _run_gather(x8, sg, rg):
    i32 = jnp.int32
    f32 = jnp.float32
    return pl.kernel(
        _gather_body,
        out_type=(jax.ShapeDtypeStruct((EP, 8), f32),
                  jax.ShapeDtypeStruct((EP, 8), f32)),
        mesh=_sc_mesh,
        compiler_params=_sc_params,
        scratch_types=[
            pltpu.VMEM((GCH,), i32), pltpu.VMEM((GCH,), i32),
            pltpu.VMEM((GCH, 8), f32), pltpu.VMEM((GCH, 8), f32),
            pltpu.VMEM((GCH,), i32), pltpu.VMEM((GCH,), i32),
            pltpu.VMEM((GCH, 8), f32), pltpu.VMEM((GCH, 8), f32),
            pltpu.SemaphoreType.DMA,
            pltpu.SemaphoreType.DMA,
            pltpu.SemaphoreType.DMA,
        ],
    )(x8, sg, rg)


def _run_relation_mlp(spos, rpos, rm_w1, rm_b1, rm_w2, rm_b2, rm_w3, rm_b3,
                      rm_w4, rm_b4):
    f32 = jnp.float32
    bf16 = jnp.bfloat16
    w1s = jnp.pad(rm_w1[0:2], ((0, 6), (0, 0)))
    w1r = jnp.pad(rm_w1[2:4], ((0, 6), (0, 0)))
    w2b = rm_w2.astype(bf16)
    w3b = rm_w3.astype(bf16)
    w4p = jnp.pad(rm_w4, ((0, 0), (0, HIDP - HID))).astype(bf16)
    b4p = jnp.pad(rm_b4, (0, HIDP - HID))
    return pl.pallas_call(
        _relation_mlp_body,
        grid=(E_GRID,),
        in_specs=[
            pl.BlockSpec((ET, 8), lambda i: (i, 0)),
            pl.BlockSpec((ET, 8), lambda i: (i, 0)),
            _full((8, HID)), _full((8, HID)), _full((1, HID)),
            _full((HID, HID)), _full((1, HID)),
            _full((HID, HID)), _full((1, HID)),
            _full((HID, HIDP)), _full((1, HIDP)),
        ],
        out_specs=pl.BlockSpec((ET, HIDP), lambda i: (i, 0)),
        out_shape=jax.ShapeDtypeStruct((EP, HIDP), f32),
        compiler_params=pltpu.CompilerParams(
            dimension_semantics=("arbitrary",)),
    )(spos, rpos, w1s, w1r, rm_b1.reshape(1, HID),
      w2b, rm_b2.reshape(1, HID), w3b, rm_b3.reshape(1, HID),
      w4p, b4p.reshape(1, HIDP))


def _run_scatter(e, ss, zeros):
    i32 = jnp.int32
    f32 = jnp.float32
    return pl.kernel(
        _scatter_body,
        out_type=jax.ShapeDtypeStruct((N, HIDP), f32),
        mesh=_sc_mesh,
        compiler_params=_sc_params,
        scratch_types=[
            pltpu.VMEM_SHARED((NPAD, Q), f32),
            pltpu.VMEM((SK, 128), i32), pltpu.VMEM((SCH, Q), f32),
            pltpu.VMEM((SK, 128), i32), pltpu.VMEM((SCH, Q), f32),
            pltpu.SemaphoreType.DMA,
            pltpu.SemaphoreType.DMA,
        ],
    )(e, ss, zeros)


def _run_object_mlp(x, agg, om_w1, om_b1, om_w2, om_b2):
    f32 = jnp.float32
    w1a = jnp.pad(om_w1[2:], ((0, HIDP - HID), (0, 0)))
    return pl.pallas_call(
        _object_mlp_body,
        grid=(N_GRID,),
        in_specs=[
            pl.BlockSpec((NTC, 2), lambda i: (i, 0)),
            pl.BlockSpec((NTC, HIDP), lambda i: (i, 0)),
            _full((2, HID)), _full((HIDP, HID)), _full((1, HID)),
            _full((HID, 2)), _full((1, 2)),
        ],
        out_specs=pl.BlockSpec((NTC, 2), lambda i: (i, 0)),
        out_shape=jax.ShapeDtypeStruct((N, 2), f32),
        compiler_params=pltpu.CompilerParams(
            dimension_semantics=("arbitrary",)),
    )(x, agg, om_w1[0:2], w1a, om_b1.reshape(1, HID),
      om_w2, om_b2.reshape(1, 2))


def _prep_indices(x, edge_index):
    sender = edge_index[0]
    receiver = edge_index[1]
    pad = EP - E
    sg = jnp.pad(sender, (0, pad))
    rg = jnp.pad(receiver, (0, pad))
    ss = jnp.pad(sender, (0, pad), constant_values=N).reshape(ROWS, 128)
    x8 = jnp.pad(x, ((0, 0), (0, 6)))
    zeros = jnp.zeros((NT, Q), jnp.float32)
    return x8, sg, rg, ss, zeros


def kernel(x, edge_index, rm_w1, rm_b1, rm_w2, rm_b2, rm_w3, rm_b3,
           rm_w4, rm_b4, om_w1, om_b1, om_w2, om_b2):
    x8, sg, rg, ss, zeros = _prep_indices(x, edge_index)
    spos, rpos = _run_gather(x8, sg, rg)
    e = _run_relation_mlp(spos, rpos, rm_w1, rm_b1, rm_w2, rm_b2,
                          rm_w3, rm_b3, rm_w4, rm_b4)
    agg = _run_scatter(e, ss, zeros)
    return _run_object_mlp(x, agg, om_w1, om_b1, om_w2, om_b2)


# speedup vs baseline: 1.0014x; 1.0014x over previous
import jax
import jax.numpy as jnp
from jax import lax
from jax.experimental import pallas as pl
from jax.experimental.pallas import tpu as pltpu
from jax.experimental.pallas import tpu_sc as plsc

N = 50000
E = 800000
EP = 802816
ROWS = EP // 128
HID = 100
HIDP = 128

NW = 32
W_EDGES = EP // NW
GCH = 3136
GW = GCH // 16
G_NCHUNK = W_EDGES // GCH

T_EDGES = EP // 16
SCH = 256
S_NCHUNK = T_EDGES // SCH
NPAD = 50008
NT = N // 16
Q = 32

_sc_mesh = plsc.VectorSubcoreMesh(core_axis_name="c", subcore_axis_name="s")
_sc_params = pltpu.CompilerParams(use_tc_tiling_on_sc=False)


def _gather_body(x8, sg, rg, spos, rpos,
                 sidx0, ridx0, srows0, rrows0,
                 sidx1, ridx1, srows1, rrows1,
                 isem, gsem, wsem):
    c = lax.axis_index("c")
    s = lax.axis_index("s")
    wid = s * 2 + c
    edge_base = wid * W_EDGES
    bufs = ((sidx0, ridx0, srows0, rrows0), (sidx1, ridx1, srows1, rrows1))

    def load_idx(i, b):
        sidx, ridx, _, _ = bufs[b]
        e0 = edge_base + i * GCH
        pltpu.async_copy(sg.at[pl.ds(e0, GCH)], sidx, isem)
        pltpu.async_copy(rg.at[pl.ds(e0, GCH)], ridx, isem)

    def drain_idx(b):
        sidx, ridx, _, _ = bufs[b]
        pltpu.make_async_copy(sg.at[pl.ds(0, GCH)], sidx, isem).wait()
        pltpu.make_async_copy(rg.at[pl.ds(0, GCH)], ridx, isem).wait()

    def fire_gather(b):
        sidx, ridx, srows, rrows = bufs[b]
        pltpu.async_copy(x8.at[sidx], srows, gsem)
        pltpu.async_copy(x8.at[ridx], rrows, gsem)

    def drain_gather(b):
        sidx, ridx, srows, rrows = bufs[b]
        pltpu.make_async_copy(x8.at[sidx], srows, gsem).wait()
        pltpu.make_async_copy(x8.at[ridx], rrows, gsem).wait()

    def fire_wb(i, b):
        _, _, srows, rrows = bufs[b]
        e0 = edge_base + i * GCH
        pltpu.async_copy(srows, spos.at[pl.ds(e0, GCH)], wsem)
        pltpu.async_copy(rrows, rpos.at[pl.ds(e0, GCH)], wsem)

    def drain_wb(b):
        _, _, srows, rrows = bufs[b]
        pltpu.make_async_copy(srows, spos.at[pl.ds(0, GCH)], wsem).wait()
        pltpu.make_async_copy(rrows, rpos.at[pl.ds(0, GCH)], wsem).wait()

    load_idx(0, 0)

    def pair(p, carry):
        i = p * 2
        drain_idx(0)
        load_idx(i + 1, 1)

        @pl.when(p > 0)
        def _():
            drain_wb(0)
        fire_gather(0)
        drain_idx(1)
        drain_gather(0)
        fire_wb(i, 0)

        @pl.when(p > 0)
        def _():
            drain_wb(1)
        fire_gather(1)

        @pl.when(p + 1 < G_NCHUNK // 2)
        def _():
            load_idx(i + 2, 0)
        drain_gather(1)
        fire_wb(i + 1, 1)
        return carry

    lax.fori_loop(0, G_NCHUNK // 2, pair, 0)
    drain_wb(0)
    drain_wb(1)


def _scatter_body(e, ss, zeros, agg, acc,
                  sidx0, ebuf0, sidx1, ebuf1, lsem, csem):
    c = lax.axis_index("c")
    s = lax.axis_index("s")
    bufs = ((sidx0, ebuf0), (sidx1, ebuf1))

    def do_quarter(qoff):
        pltpu.sync_copy(zeros, acc.at[pl.ds(s * NT, NT)])
        plsc.subcore_barrier()

        def load(i, b):
            sidx, ebuf = bufs[b]
            e0 = s * T_EDGES + i * SCH
            pltpu.async_copy(ss.at[pl.ds(e0, SCH)], sidx, lsem)
            pltpu.async_copy(
                e.at[pl.ds(e0, SCH), pl.ds(qoff, Q)], ebuf, lsem)

        def drain_load(b):
            sidx, ebuf = bufs[b]
            pltpu.make_async_copy(ss.at[pl.ds(0, SCH)], sidx, lsem).wait()
            pltpu.make_async_copy(
                e.at[pl.ds(0, SCH), pl.ds(qoff, Q)], ebuf, lsem).wait()

        def fire_scatter(b):
            sidx, ebuf = bufs[b]
            pltpu.async_copy(ebuf, acc.at[sidx], csem, add=True)

        def drain_scatter(b):
            sidx, ebuf = bufs[b]
            pltpu.make_async_copy(ebuf, acc.at[sidx], csem).wait()

        load(0, 0)

        def pair(p, carry):
            i = p * 2

            @pl.when(p > 0)
            def _():
                drain_scatter(1)
            load(i + 1, 1)
            drain_load(0)
            fire_scatter(0)
            drain_load(1)
            drain_scatter(0)

            @pl.when(p + 1 < S_NCHUNK // 2)
            def _():
                load(i + 2, 0)
            fire_scatter(1)
            return carry

        lax.fori_loop(0, S_NCHUNK // 2, pair, 0)
        drain_scatter(1)
        plsc.subcore_barrier()
        pltpu.sync_copy(acc.at[pl.ds(s * NT, NT)],
                        agg.at[pl.ds(s * NT, NT), pl.ds(qoff, Q)])
        plsc.subcore_barrier()

    @pl.when(c == 0)
    def _():
        do_quarter(0)
        do_quarter(2 * Q)

    @pl.when(c == 1)
    def _():
        do_quarter(Q)
        do_quarter(3 * Q)


def _relation_mlp_body(sp, rp, w1s, w1r, b1, w2, b2, w3, b3, w4, b4, out):
    f32 = jnp.float32
    bf16 = jnp.bfloat16
    h = jnp.dot(sp[...], w1s[...], preferred_element_type=f32)
    h = h + jnp.dot(rp[...], w1r[...], preferred_element_type=f32) + b1[...]
    h = jnp.maximum(h, 0.0)
    h = jnp.maximum(jnp.dot(h.astype(bf16), w2[...],
                            preferred_element_type=f32) + b2[...], 0.0)
    h = jnp.maximum(jnp.dot(h.astype(bf16), w3[...],
                            preferred_element_type=f32) + b3[...], 0.0)
    h = jnp.maximum(jnp.dot(h.astype(bf16), w4[...],
                            preferred_element_type=f32) + b4[...], 0.0)
    out[...] = h


def _object_mlp_body(x, agg, w1x, w1a, b1, w2, b2, out):
    f32 = jnp.float32
    h = jnp.dot(x[...], w1x[...], preferred_element_type=f32)
    h = h + jnp.dot(agg[...], w1a[...], preferred_element_type=f32) + b1[...]
    h = jnp.maximum(h, 0.0)
    out[...] = jnp.dot(h, w2[...], preferred_element_type=f32) + b2[...]


ET = 4096
E_GRID = EP // ET
NTC = 2000
N_GRID = N // NTC


def _full(shape):
    return pl.BlockSpec(shape, lambda i: (0,) * len(shape))


def _run_gather(x8, sg, rg):
    i32 = jnp.int32
    f32 = jnp.float32
    return pl.kernel(
        _gather_body,
        out_type=(jax.ShapeDtypeStruct((EP, 8), f32),
                  jax.ShapeDtypeStruct((EP, 8), f32)),
        mesh=_sc_mesh,
        compiler_params=_sc_params,
        scratch_types=[
            pltpu.VMEM((GCH,), i32), pltpu.VMEM((GCH,), i32),
            pltpu.VMEM((GCH, 8), f32), pltpu.VMEM((GCH, 8), f32),
            pltpu.VMEM((GCH,), i32), pltpu.VMEM((GCH,), i32),
            pltpu.VMEM((GCH, 8), f32), pltpu.VMEM((GCH, 8), f32),
            pltpu.SemaphoreType.DMA,
            pltpu.SemaphoreType.DMA,
            pltpu.SemaphoreType.DMA,
        ],
    )(x8, sg, rg)


def _run_relation_mlp(spos, rpos, rm_w1, rm_b1, rm_w2, rm_b2, rm_w3, rm_b3,
                      rm_w4, rm_b4):
    f32 = jnp.float32
    bf16 = jnp.bfloat16
    w1s = jnp.pad(rm_w1[0:2], ((0, 6), (0, 0)))
    w1r = jnp.pad(rm_w1[2:4], ((0, 6), (0, 0)))
    w2b = rm_w2.astype(bf16)
    w3b = rm_w3.astype(bf16)
    w4p = jnp.pad(rm_w4, ((0, 0), (0, HIDP - HID))).astype(bf16)
    b4p = jnp.pad(rm_b4, (0, HIDP - HID))
    return pl.pallas_call(
        _relation_mlp_body,
        grid=(E_GRID,),
        in_specs=[
            pl.BlockSpec((ET, 8), lambda i: (i, 0)),
            pl.BlockSpec((ET, 8), lambda i: (i, 0)),
            _full((8, HID)), _full((8, HID)), _full((1, HID)),
            _full((HID, HID)), _full((1, HID)),
            _full((HID, HID)), _full((1, HID)),
            _full((HID, HIDP)), _full((1, HIDP)),
        ],
        out_specs=pl.BlockSpec((ET, HIDP), lambda i: (i, 0)),
        out_shape=jax.ShapeDtypeStruct((EP, HIDP), f32),
        compiler_params=pltpu.CompilerParams(
            dimension_semantics=("arbitrary",)),
    )(spos, rpos, w1s, w1r, rm_b1.reshape(1, HID),
      w2b, rm_b2.reshape(1, HID), w3b, rm_b3.reshape(1, HID),
      w4p, b4p.reshape(1, HIDP))


def _run_scatter(e, ss, zeros):
    i32 = jnp.int32
    f32 = jnp.float32
    return pl.kernel(
        _scatter_body,
        out_type=jax.ShapeDtypeStruct((N, HIDP), f32),
        mesh=_sc_mesh,
        compiler_params=_sc_params,
        scratch_types=[
            pltpu.VMEM_SHARED((NPAD, Q), f32),
            pltpu.VMEM((SCH,), i32), pltpu.VMEM((SCH, Q), f32),
            pltpu.VMEM((SCH,), i32), pltpu.VMEM((SCH, Q), f32),
            pltpu.SemaphoreType.DMA,
            pltpu.SemaphoreType.DMA,
        ],
    )(e, ss, zeros)


def _run_object_mlp(x, agg, om_w1, om_b1, om_w2, om_b2):
    f32 = jnp.float32
    w1a = jnp.pad(om_w1[2:], ((0, HIDP - HID), (0, 0)))
    return pl.pallas_call(
        _object_mlp_body,
        grid=(N_GRID,),
        in_specs=[
            pl.BlockSpec((NTC, 2), lambda i: (i, 0)),
            pl.BlockSpec((NTC, HIDP), lambda i: (i, 0)),
            _full((2, HID)), _full((HIDP, HID)), _full((1, HID)),
            _full((HID, 2)), _full((1, 2)),
        ],
        out_specs=pl.BlockSpec((NTC, 2), lambda i: (i, 0)),
        out_shape=jax.ShapeDtypeStruct((N, 2), f32),
        compiler_params=pltpu.CompilerParams(
            dimension_semantics=("arbitrary",)),
    )(x, agg, om_w1[0:2], w1a, om_b1.reshape(1, HID),
      om_w2, om_b2.reshape(1, 2))


def _prep_indices(x, edge_index):
    sender = edge_index[0]
    receiver = edge_index[1]
    pad = EP - E
    sg = jnp.pad(sender, (0, pad))
    rg = jnp.pad(receiver, (0, pad))
    ss = jnp.pad(sender, (0, pad), constant_values=N)
    x8 = jnp.pad(x, ((0, 0), (0, 6)))
    zeros = jnp.zeros((NT, Q), jnp.float32)
    return x8, sg, rg, ss, zeros


def kernel(x, edge_index, rm_w1, rm_b1, rm_w2, rm_b2, rm_w3, rm_b3,
           rm_w4, rm_b4, om_w1, om_b1, om_w2, om_b2):
    x8, sg, rg, ss, zeros = _prep_indices(x, edge_index)
    spos, rpos = _run_gather(x8, sg, rg)
    e = _run_relation_mlp(spos, rpos, rm_w1, rm_b1, rm_w2, rm_b2,
                          rm_w3, rm_b3, rm_w4, rm_b4)
    agg = _run_scatter(e, ss, zeros)
    return _run_object_mlp(x, agg, om_w1, om_b1, om_w2, om_b2)
